# baseline (speedup 1.0000x reference)
import functools

import jax
import jax.numpy as jnp
from jax import lax
from jax.experimental import pallas as pl
from jax.experimental.pallas import tpu as pltpu
from jax.experimental.pallas import tpu_sc as plsc

HIDDEN = 768
EDGE_FEAT = 9
MAX_PANELS = 37
MAX_EDGES = 39
NUM_STITCH = 8
CAT = 96
PAD_T = 1456
CHUNK = 16
NW = 32


def _sc_gather_call(tab, pi, ei, si, B):
    n_chunks = B * (PAD_T // CHUNK)
    per_w = n_chunks // NW
    mesh = plsc.VectorSubcoreMesh(core_axis_name="c", subcore_axis_name="s")

    @functools.partial(
        pl.kernel, mesh=mesh,
        compiler_params=pltpu.CompilerParams(needs_layout_passes=False),
        out_type=jax.ShapeDtypeStruct((B, PAD_T, HIDDEN), jnp.float32),
        scratch_types=[
            pltpu.VMEM((CAT, HIDDEN), jnp.float32),
            pltpu.VMEM((CHUNK,), jnp.int32),
            pltpu.VMEM((CHUNK,), jnp.int32),
            pltpu.VMEM((CHUNK,), jnp.int32),
            pltpu.VMEM((CHUNK, HIDDEN), jnp.float32),
        ],
    )
    def k(tab_hbm, pi_hbm, ei_hbm, si_hbm, g_hbm,
          tab_v, pi_v, ei_v, si_v, gbuf_v):
        wid = lax.axis_index("s") * 2 + lax.axis_index("c")
        pltpu.sync_copy(tab_hbm, tab_v)
        rel = lax.iota(jnp.int32, CHUNK)

        def chunk_body(l, _):
            c = wid * per_w + l
            b = c // (PAD_T // CHUNK)
            tok0 = (c % (PAD_T // CHUNK)) * CHUNK
            pltpu.sync_copy(pi_hbm.at[b, pl.ds(tok0, CHUNK)], pi_v)
            pltpu.sync_copy(ei_hbm.at[b, pl.ds(tok0, CHUNK)], ei_v)
            pltpu.sync_copy(si_hbm.at[b, pl.ds(tok0, CHUNK)], si_v)
            pv, ev, sv = pi_v[...], ei_v[...], si_v[...]

            def col_body(j, _):
                cj = jnp.full((CHUNK,), 0, jnp.int32) + j
                g = (plsc.load_gather(tab_v, [pv, cj])
                     + plsc.load_gather(tab_v, [ev, cj])
                     + plsc.load_gather(tab_v, [sv, cj]))
                plsc.store_scatter(gbuf_v, [rel, cj], g)
                return 0

            lax.fori_loop(0, HIDDEN, col_body, 0)
            pltpu.sync_copy(gbuf_v, g_hbm.at[b, pl.ds(tok0, CHUNK)])
            return 0

        lax.fori_loop(0, per_w, chunk_body, 0)

    return k(tab, pi, ei, si)


def _tc_body(ep_ref, g_ref, w_ref, b_ref, gam_ref, beta_ref, out_ref):
    BB, T = ep_ref.shape[0], ep_ref.shape[1]
    for bb in range(BB):
        ep = ep_ref[bb]
        acc = jnp.dot(ep, w_ref[...], preferred_element_type=jnp.float32)
        acc = acc + b_ref[...] + g_ref[bb, :T, :]
        mean = jnp.mean(acc, axis=1, keepdims=True)
        cen = acc - mean
        var = jnp.mean(cen * cen, axis=1, keepdims=True)
        inv = lax.rsqrt(var + 1e-5)
        out_ref[bb] = cen * inv * gam_ref[...] + beta_ref[...]


def kernel(edge_parameters, stitch_types, panel_indices, edge_indices,
           W_edge, b_edge, panel_tab, edge_tab, stitch_tab, ln_gamma, ln_beta):
    B, P, E, F = edge_parameters.shape
    T = P * E
    ep = edge_parameters.reshape(B, T, F)
    pad = ((0, 0), (0, PAD_T - T))
    pi = jnp.pad(panel_indices.reshape(B, T).astype(jnp.int32), pad)
    ei = jnp.pad(edge_indices.reshape(B, T).astype(jnp.int32) + MAX_PANELS, pad)
    si = jnp.pad(stitch_types.reshape(B, T).astype(jnp.int32)
                 + (MAX_PANELS + MAX_EDGES), pad)
    tab = jnp.concatenate(
        [panel_tab, edge_tab, stitch_tab,
         jnp.zeros((CAT - MAX_PANELS - MAX_EDGES - NUM_STITCH, HIDDEN),
                   jnp.float32)], axis=0)

    g = _sc_gather_call(tab, pi, ei, si, B)

    BB = 2
    tok_spec = pl.BlockSpec((BB, T, EDGE_FEAT), lambda i: (i, 0, 0))
    g_spec = pl.BlockSpec((BB, PAD_T, HIDDEN), lambda i: (i, 0, 0))
    full = lambda shape: pl.BlockSpec(shape, lambda i: (0,) * len(shape))
    out = pl.pallas_call(
        _tc_body,
        grid=(B // BB,),
        in_specs=[
            tok_spec, g_spec,
            full((EDGE_FEAT, HIDDEN)),
            full((1, HIDDEN)),
            full((1, HIDDEN)),
            full((1, HIDDEN)),
        ],
        out_specs=pl.BlockSpec((BB, T, HIDDEN), lambda i: (i, 0, 0)),
        out_shape=jax.ShapeDtypeStruct((B, T, HIDDEN), jnp.float32),
    )(ep, g, W_edge, b_edge.reshape(1, HIDDEN),
      ln_gamma.reshape(1, HIDDEN), ln_beta.reshape(1, HIDDEN))
    return out


# speedup vs baseline: 1.0508x; 1.0508x over previous
import functools

import jax
import jax.numpy as jnp
from jax import lax
from jax.experimental import pallas as pl
from jax.experimental.pallas import tpu as pltpu
from jax.experimental.pallas import tpu_sc as plsc

HIDDEN = 768
EDGE_FEAT = 9
MAX_PANELS = 37
MAX_EDGES = 39
NUM_STITCH = 8
CAT = 96
PAD_T = 1456
CHUNK = 16
NW = 32


def _sc_gather_call(tab, pi, ei, si, B):
    n_chunks = B * (PAD_T // CHUNK)
    per_w = n_chunks // NW
    mesh = plsc.VectorSubcoreMesh(core_axis_name="c", subcore_axis_name="s")

    n_per_b = PAD_T // CHUNK
    b_per_w = B // NW

    @functools.partial(
        pl.kernel, mesh=mesh,
        compiler_params=pltpu.CompilerParams(needs_layout_passes=False,
                                             disable_bounds_checks=True),
        out_type=jax.ShapeDtypeStruct((B, PAD_T, HIDDEN), jnp.float32),
        scratch_types=[
            pltpu.VMEM((CAT, HIDDEN), jnp.float32),
            pltpu.VMEM((PAD_T,), jnp.int32),
            pltpu.VMEM((PAD_T,), jnp.int32),
            pltpu.VMEM((PAD_T,), jnp.int32),
            pltpu.VMEM((2, CHUNK, HIDDEN), jnp.float32),
            pltpu.SemaphoreType.DMA((2,)),
        ],
    )
    def k(tab_hbm, pi_hbm, ei_hbm, si_hbm, g_hbm,
          tab_v, pi_v, ei_v, si_v, gbuf_v, sem):
        wid = lax.axis_index("s") * 2 + lax.axis_index("c")
        pltpu.sync_copy(tab_hbm, tab_v)
        rel = lax.iota(jnp.int32, CHUNK)

        for bb in range(b_per_w):
            b = wid * b_per_w + bb
            pltpu.sync_copy(pi_hbm.at[b], pi_v)
            pltpu.sync_copy(ei_hbm.at[b], ei_v)
            pltpu.sync_copy(si_hbm.at[b], si_v)

            def chunk_body(l, _, b=b):
                tok0 = l * CHUNK
                slot = lax.rem(l, 2)
                pv = pi_v[pl.ds(tok0, CHUNK)]
                ev = ei_v[pl.ds(tok0, CHUNK)]
                sv = si_v[pl.ds(tok0, CHUNK)]

                @pl.when(l >= 2)
                def _():
                    pltpu.make_async_copy(
                        gbuf_v.at[slot],
                        g_hbm.at[b, pl.ds(tok0, CHUNK)], sem.at[slot]).wait()

                def col_body(jg, _):
                    j0 = jg * CHUNK
                    for j2 in range(CHUNK):
                        cj = jnp.full((CHUNK,), j2, jnp.int32) + j0
                        g = (plsc.load_gather(tab_v, [pv, cj])
                             + plsc.load_gather(tab_v, [ev, cj])
                             + plsc.load_gather(tab_v, [sv, cj]))
                        plsc.store_scatter(gbuf_v.at[slot], [rel, cj], g)
                    return 0

                lax.fori_loop(0, HIDDEN // CHUNK, col_body, 0)
                pltpu.make_async_copy(gbuf_v.at[slot],
                                      g_hbm.at[b, pl.ds(tok0, CHUNK)],
                                      sem.at[slot]).start()
                return 0

            lax.fori_loop(0, n_per_b, chunk_body, 0)
            for tail in (n_per_b - 2, n_per_b - 1):
                pltpu.make_async_copy(
                    gbuf_v.at[tail % 2],
                    g_hbm.at[b, pl.ds(tail * CHUNK, CHUNK)],
                    sem.at[tail % 2]).wait()

    return k(tab, pi, ei, si)


def _tc_body(ep_ref, g_ref, w_ref, b_ref, gam_ref, beta_ref, out_ref):
    BB, T = ep_ref.shape[0], ep_ref.shape[1]
    for bb in range(BB):
        ep = ep_ref[bb]
        acc = jnp.dot(ep, w_ref[...], preferred_element_type=jnp.float32)
        acc = acc + b_ref[...] + g_ref[bb, :T, :]
        mean = jnp.mean(acc, axis=1, keepdims=True)
        cen = acc - mean
        var = jnp.mean(cen * cen, axis=1, keepdims=True)
        inv = lax.rsqrt(var + 1e-5)
        out_ref[bb] = cen * inv * gam_ref[...] + beta_ref[...]


def kernel(edge_parameters, stitch_types, panel_indices, edge_indices,
           W_edge, b_edge, panel_tab, edge_tab, stitch_tab, ln_gamma, ln_beta):
    B, P, E, F = edge_parameters.shape
    T = P * E
    ep = edge_parameters.reshape(B, T, F)
    pad = ((0, 0), (0, PAD_T - T))
    pi = jnp.pad(panel_indices.reshape(B, T).astype(jnp.int32), pad)
    ei = jnp.pad(edge_indices.reshape(B, T).astype(jnp.int32) + MAX_PANELS, pad)
    si = jnp.pad(stitch_types.reshape(B, T).astype(jnp.int32)
                 + (MAX_PANELS + MAX_EDGES), pad)
    tab = jnp.concatenate(
        [panel_tab, edge_tab, stitch_tab,
         jnp.zeros((CAT - MAX_PANELS - MAX_EDGES - NUM_STITCH, HIDDEN),
                   jnp.float32)], axis=0)

    g = _sc_gather_call(tab, pi, ei, si, B)

    BB = 2
    tok_spec = pl.BlockSpec((BB, T, EDGE_FEAT), lambda i: (i, 0, 0))
    g_spec = pl.BlockSpec((BB, PAD_T, HIDDEN), lambda i: (i, 0, 0))
    full = lambda shape: pl.BlockSpec(shape, lambda i: (0,) * len(shape))
    out = pl.pallas_call(
        _tc_body,
        grid=(B // BB,),
        in_specs=[
            tok_spec, g_spec,
            full((EDGE_FEAT, HIDDEN)),
            full((1, HIDDEN)),
            full((1, HIDDEN)),
            full((1, HIDDEN)),
        ],
        out_specs=pl.BlockSpec((BB, T, HIDDEN), lambda i: (i, 0, 0)),
        out_shape=jax.ShapeDtypeStruct((B, T, HIDDEN), jnp.float32),
    )(ep, g, W_edge, b_edge.reshape(1, HIDDEN),
      ln_gamma.reshape(1, HIDDEN), ln_beta.reshape(1, HIDDEN))
    return out


# speedup vs baseline: 1.3538x; 1.2884x over previous
import functools

import jax
import jax.numpy as jnp
from jax import lax
from jax.experimental import pallas as pl
from jax.experimental.pallas import tpu as pltpu
from jax.experimental.pallas import tpu_sc as plsc

HIDDEN = 768
EDGE_FEAT = 9
MAX_PANELS = 37
MAX_EDGES = 39
NUM_STITCH = 8
CAT = 96
PAD_T = 1456
CHUNK = 16
NW = 32


def _sc_gather_call(tab, pi, ei, si, B):
    n_chunks = B * (PAD_T // CHUNK)
    per_w = n_chunks // NW
    mesh = plsc.VectorSubcoreMesh(core_axis_name="c", subcore_axis_name="s")

    n_per_b = PAD_T // CHUNK
    b_per_w = B // NW

    @functools.partial(
        pl.kernel, mesh=mesh,
        compiler_params=pltpu.CompilerParams(needs_layout_passes=False,
                                             disable_bounds_checks=True),
        out_type=jax.ShapeDtypeStruct((B, PAD_T, HIDDEN), jnp.float32),
        scratch_types=[
            pltpu.VMEM((CAT, HIDDEN), jnp.float32),
            pltpu.VMEM((PAD_T,), jnp.int32),
            pltpu.VMEM((PAD_T,), jnp.int32),
            pltpu.VMEM((PAD_T,), jnp.int32),
            pltpu.VMEM((2, CHUNK, HIDDEN), jnp.float32),
            pltpu.SemaphoreType.DMA((2,)),
        ],
    )
    def k(tab_hbm, pi_hbm, ei_hbm, si_hbm, g_hbm,
          tab_v, pi_v, ei_v, si_v, gbuf_v, sem):
        wid = lax.axis_index("s") * 2 + lax.axis_index("c")
        pltpu.sync_copy(tab_hbm, tab_v)
        rel = lax.iota(jnp.int32, CHUNK)

        for bb in range(b_per_w):
            b = wid * b_per_w + bb
            pltpu.sync_copy(pi_hbm.at[b], pi_v)
            pltpu.sync_copy(ei_hbm.at[b], ei_v)
            pltpu.sync_copy(si_hbm.at[b], si_v)

            def chunk_body(l, _, b=b):
                tok0 = l * CHUNK
                slot = lax.rem(l, 2)
                pv = pi_v[pl.ds(tok0, CHUNK)]
                ev = ei_v[pl.ds(tok0, CHUNK)]
                sv = si_v[pl.ds(tok0, CHUNK)]

                @pl.when(l >= 2)
                def _():
                    pltpu.make_async_copy(
                        gbuf_v.at[slot],
                        g_hbm.at[b, pl.ds(tok0, CHUNK)], sem.at[slot]).wait()

                @plsc.parallel_loop(0, HIDDEN // CHUNK, 1, unroll=2)
                def col_body(jg):
                    j0 = jg * CHUNK
                    for j2 in range(CHUNK):
                        cj = jnp.full((CHUNK,), j2, jnp.int32) + j0
                        g = (plsc.load_gather(tab_v, [pv, cj])
                             + plsc.load_gather(tab_v, [ev, cj])
                             + plsc.load_gather(tab_v, [sv, cj]))
                        plsc.store_scatter(gbuf_v.at[slot], [rel, cj], g)
                pltpu.make_async_copy(gbuf_v.at[slot],
                                      g_hbm.at[b, pl.ds(tok0, CHUNK)],
                                      sem.at[slot]).start()
                return 0

            lax.fori_loop(0, n_per_b, chunk_body, 0)
            for tail in (n_per_b - 2, n_per_b - 1):
                pltpu.make_async_copy(
                    gbuf_v.at[tail % 2],
                    g_hbm.at[b, pl.ds(tail * CHUNK, CHUNK)],
                    sem.at[tail % 2]).wait()

    return k(tab, pi, ei, si)


def _tc_body(ep_ref, g_ref, w_ref, b_ref, gam_ref, beta_ref, out_ref):
    BB, T = ep_ref.shape[0], ep_ref.shape[1]
    for bb in range(BB):
        ep = ep_ref[bb]
        acc = jnp.dot(ep, w_ref[...], preferred_element_type=jnp.float32)
        acc = acc + b_ref[...] + g_ref[bb, :T, :]
        mean = jnp.mean(acc, axis=1, keepdims=True)
        cen = acc - mean
        var = jnp.mean(cen * cen, axis=1, keepdims=True)
        inv = lax.rsqrt(var + 1e-5)
        out_ref[bb] = cen * inv * gam_ref[...] + beta_ref[...]


def kernel(edge_parameters, stitch_types, panel_indices, edge_indices,
           W_edge, b_edge, panel_tab, edge_tab, stitch_tab, ln_gamma, ln_beta):
    B, P, E, F = edge_parameters.shape
    T = P * E
    ep = edge_parameters.reshape(B, T, F)
    pad = ((0, 0), (0, PAD_T - T))
    pi = jnp.pad(panel_indices.reshape(B, T).astype(jnp.int32), pad)
    ei = jnp.pad(edge_indices.reshape(B, T).astype(jnp.int32) + MAX_PANELS, pad)
    si = jnp.pad(stitch_types.reshape(B, T).astype(jnp.int32)
                 + (MAX_PANELS + MAX_EDGES), pad)
    tab = jnp.concatenate(
        [panel_tab, edge_tab, stitch_tab,
         jnp.zeros((CAT - MAX_PANELS - MAX_EDGES - NUM_STITCH, HIDDEN),
                   jnp.float32)], axis=0)

    g = _sc_gather_call(tab, pi, ei, si, B)

    BB = 2
    tok_spec = pl.BlockSpec((BB, T, EDGE_FEAT), lambda i: (i, 0, 0))
    g_spec = pl.BlockSpec((BB, PAD_T, HIDDEN), lambda i: (i, 0, 0))
    full = lambda shape: pl.BlockSpec(shape, lambda i: (0,) * len(shape))
    out = pl.pallas_call(
        _tc_body,
        grid=(B // BB,),
        in_specs=[
            tok_spec, g_spec,
            full((EDGE_FEAT, HIDDEN)),
            full((1, HIDDEN)),
            full((1, HIDDEN)),
            full((1, HIDDEN)),
        ],
        out_specs=pl.BlockSpec((BB, T, HIDDEN), lambda i: (i, 0, 0)),
        out_shape=jax.ShapeDtypeStruct((B, T, HIDDEN), jnp.float32),
    )(ep, g, W_edge, b_edge.reshape(1, HIDDEN),
      ln_gamma.reshape(1, HIDDEN), ln_beta.reshape(1, HIDDEN))
    return out


# speedup vs baseline: 1.3544x; 1.0004x over previous
import functools

import jax
import jax.numpy as jnp
from jax import lax
from jax.experimental import pallas as pl
from jax.experimental.pallas import tpu as pltpu
from jax.experimental.pallas import tpu_sc as plsc

HIDDEN = 768
EDGE_FEAT = 9
MAX_PANELS = 37
MAX_EDGES = 39
NUM_STITCH = 8
CAT = 96
PAD_T = 1456
CHUNK = 16
NW = 32


def _sc_gather_call(tab, pi, ei, si, B):
    n_chunks = B * (PAD_T // CHUNK)
    per_w = n_chunks // NW
    mesh = plsc.VectorSubcoreMesh(core_axis_name="c", subcore_axis_name="s")

    n_per_b = PAD_T // CHUNK
    b_per_w = B // NW

    @functools.partial(
        pl.kernel, mesh=mesh,
        compiler_params=pltpu.CompilerParams(needs_layout_passes=False,
                                             disable_bounds_checks=True),
        out_type=jax.ShapeDtypeStruct((B, PAD_T, HIDDEN), jnp.float32),
        scratch_types=[
            pltpu.VMEM((CAT, HIDDEN + 1), jnp.float32),
            pltpu.VMEM((PAD_T,), jnp.int32),
            pltpu.VMEM((PAD_T,), jnp.int32),
            pltpu.VMEM((PAD_T,), jnp.int32),
            pltpu.VMEM((2, CHUNK, HIDDEN + 1), jnp.float32),
            pltpu.SemaphoreType.DMA((2,)),
        ],
    )
    def k(tab_hbm, pi_hbm, ei_hbm, si_hbm, g_hbm,
          tab_v, pi_v, ei_v, si_v, gbuf_v, sem):
        wid = lax.axis_index("s") * 2 + lax.axis_index("c")
        pltpu.sync_copy(tab_hbm, tab_v.at[:, pl.ds(0, HIDDEN)])
        rel = lax.iota(jnp.int32, CHUNK)

        for bb in range(b_per_w):
            b = wid * b_per_w + bb
            pltpu.sync_copy(pi_hbm.at[b], pi_v)
            pltpu.sync_copy(ei_hbm.at[b], ei_v)
            pltpu.sync_copy(si_hbm.at[b], si_v)

            def chunk_body(l, _, b=b):
                tok0 = l * CHUNK
                slot = lax.rem(l, 2)
                pv = pi_v[pl.ds(tok0, CHUNK)]
                ev = ei_v[pl.ds(tok0, CHUNK)]
                sv = si_v[pl.ds(tok0, CHUNK)]

                @pl.when(l >= 2)
                def _():
                    pltpu.make_async_copy(
                        gbuf_v.at[slot, :, pl.ds(0, HIDDEN)],
                        g_hbm.at[b, pl.ds(tok0, CHUNK)], sem.at[slot]).wait()

                @plsc.parallel_loop(0, HIDDEN // CHUNK, 1, unroll=2)
                def col_body(jg):
                    j0 = jg * CHUNK
                    for j2 in range(CHUNK):
                        cj = jnp.full((CHUNK,), j2, jnp.int32) + j0
                        g = (plsc.load_gather(tab_v, [pv, cj])
                             + plsc.load_gather(tab_v, [ev, cj])
                             + plsc.load_gather(tab_v, [sv, cj]))
                        plsc.store_scatter(gbuf_v.at[slot], [rel, cj], g)
                pltpu.make_async_copy(gbuf_v.at[slot, :, pl.ds(0, HIDDEN)],
                                      g_hbm.at[b, pl.ds(tok0, CHUNK)],
                                      sem.at[slot]).start()
                return 0

            lax.fori_loop(0, n_per_b, chunk_body, 0)
            for tail in (n_per_b - 2, n_per_b - 1):
                pltpu.make_async_copy(
                    gbuf_v.at[tail % 2, :, pl.ds(0, HIDDEN)],
                    g_hbm.at[b, pl.ds(tail * CHUNK, CHUNK)],
                    sem.at[tail % 2]).wait()

    return k(tab, pi, ei, si)


def _tc_body(ep_ref, g_ref, w_ref, b_ref, gam_ref, beta_ref, out_ref):
    BB, T = ep_ref.shape[0], ep_ref.shape[1]
    for bb in range(BB):
        ep = ep_ref[bb]
        acc = jnp.dot(ep, w_ref[...], preferred_element_type=jnp.float32)
        acc = acc + b_ref[...] + g_ref[bb, :T, :]
        mean = jnp.mean(acc, axis=1, keepdims=True)
        cen = acc - mean
        var = jnp.mean(cen * cen, axis=1, keepdims=True)
        inv = lax.rsqrt(var + 1e-5)
        out_ref[bb] = cen * inv * gam_ref[...] + beta_ref[...]


def kernel(edge_parameters, stitch_types, panel_indices, edge_indices,
           W_edge, b_edge, panel_tab, edge_tab, stitch_tab, ln_gamma, ln_beta):
    B, P, E, F = edge_parameters.shape
    T = P * E
    ep = edge_parameters.reshape(B, T, F)
    pad = ((0, 0), (0, PAD_T - T))
    pi = jnp.pad(panel_indices.reshape(B, T).astype(jnp.int32), pad)
    ei = jnp.pad(edge_indices.reshape(B, T).astype(jnp.int32) + MAX_PANELS, pad)
    si = jnp.pad(stitch_types.reshape(B, T).astype(jnp.int32)
                 + (MAX_PANELS + MAX_EDGES), pad)
    tab = jnp.concatenate(
        [panel_tab, edge_tab, stitch_tab,
         jnp.zeros((CAT - MAX_PANELS - MAX_EDGES - NUM_STITCH, HIDDEN),
                   jnp.float32)], axis=0)

    g = _sc_gather_call(tab, pi, ei, si, B)

    BB = 2
    tok_spec = pl.BlockSpec((BB, T, EDGE_FEAT), lambda i: (i, 0, 0))
    g_spec = pl.BlockSpec((BB, PAD_T, HIDDEN), lambda i: (i, 0, 0))
    full = lambda shape: pl.BlockSpec(shape, lambda i: (0,) * len(shape))
    out = pl.pallas_call(
        _tc_body,
        grid=(B // BB,),
        in_specs=[
            tok_spec, g_spec,
            full((EDGE_FEAT, HIDDEN)),
            full((1, HIDDEN)),
            full((1, HIDDEN)),
            full((1, HIDDEN)),
        ],
        out_specs=pl.BlockSpec((BB, T, HIDDEN), lambda i: (i, 0, 0)),
        out_shape=jax.ShapeDtypeStruct((B, T, HIDDEN), jnp.float32),
    )(ep, g, W_edge, b_edge.reshape(1, HIDDEN),
      ln_gamma.reshape(1, HIDDEN), ln_beta.reshape(1, HIDDEN))
    return out


# speedup vs baseline: 11.6036x; 8.5674x over previous
import jax
import jax.numpy as jnp
from jax import lax
from jax.experimental import pallas as pl

HIDDEN = 768
EDGE_FEAT = 9
MAX_PANELS = 37
MAX_EDGES = 39
NUM_STITCH = 8
CAT = 96


def _body(ep_ref, pidx_ref, eidx_ref, sidx_ref, w_ref, b_ref, tab_ref,
          g_ref, beta_ref, out_ref):
    BB, T = ep_ref.shape[0], ep_ref.shape[1]
    for bb in range(BB):
        ep = ep_ref[bb]
        acc = jnp.dot(ep, w_ref[...], preferred_element_type=jnp.float32)
        acc = acc + b_ref[...]

        p = pidx_ref[bb]
        e = eidx_ref[bb] + MAX_PANELS
        s = sidx_ref[bb] + (MAX_PANELS + MAX_EDGES)
        cols = lax.broadcasted_iota(jnp.int32, (T, CAT), 1)
        oh = ((cols == p).astype(jnp.float32)
              + (cols == e).astype(jnp.float32)
              + (cols == s).astype(jnp.float32))
        acc = acc + jnp.dot(oh, tab_ref[...], preferred_element_type=jnp.float32)

        mean = jnp.mean(acc, axis=1, keepdims=True)
        cen = acc - mean
        var = jnp.mean(cen * cen, axis=1, keepdims=True)
        inv = lax.rsqrt(var + 1e-5)
        out_ref[bb] = cen * inv * g_ref[...] + beta_ref[...]


def kernel(edge_parameters, stitch_types, panel_indices, edge_indices,
           W_edge, b_edge, panel_tab, edge_tab, stitch_tab, ln_gamma, ln_beta):
    B, P, E, F = edge_parameters.shape
    T = P * E
    ep = edge_parameters.reshape(B, T, F)
    pidx = panel_indices.reshape(B, T, 1).astype(jnp.int32)
    eidx = edge_indices.reshape(B, T, 1).astype(jnp.int32)
    sidx = stitch_types.reshape(B, T, 1).astype(jnp.int32)
    tab = jnp.concatenate(
        [panel_tab, edge_tab, stitch_tab,
         jnp.zeros((CAT - MAX_PANELS - MAX_EDGES - NUM_STITCH, HIDDEN),
                   jnp.float32)], axis=0)

    BB = 2
    tok_spec = pl.BlockSpec((BB, T, EDGE_FEAT), lambda i: (i, 0, 0))
    idx_spec = pl.BlockSpec((BB, T, 1), lambda i: (i, 0, 0))
    full = lambda shape: pl.BlockSpec(shape, lambda i: (0,) * len(shape))
    out = pl.pallas_call(
        _body,
        grid=(B // BB,),
        in_specs=[
            tok_spec, idx_spec, idx_spec, idx_spec,
            full((EDGE_FEAT, HIDDEN)),
            full((1, HIDDEN)),
            full((CAT, HIDDEN)),
            full((1, HIDDEN)),
            full((1, HIDDEN)),
        ],
        out_specs=pl.BlockSpec((BB, T, HIDDEN), lambda i: (i, 0, 0)),
        out_shape=jax.ShapeDtypeStruct((B, T, HIDDEN), jnp.float32),
    )(ep, pidx, eidx, sidx, W_edge, b_edge.reshape(1, HIDDEN), tab,
      ln_gamma.reshape(1, HIDDEN), ln_beta.reshape(1, HIDDEN))
    return out
